# baseline (speedup 1.0000x reference)
import functools

import jax
import jax.numpy as jnp
from jax import lax
from jax.experimental import pallas as pl
from jax.experimental.pallas import tpu as pltpu
from jax.experimental.pallas import tpu_sc as plsc

_L = 16


def kernel(x, table):
    (B,) = x.shape
    V, D = table.shape
    info = plsc.get_sparse_core_info()
    NC, NS = info.num_cores, info.num_subcores
    NW = NC * NS
    RANGE = 1 << 15
    CHUNK = 1024
    NCHUNK = RANGE // CHUNK
    RSHIFT = 15
    CSHIFT = 10
    GROUP = _L
    LAST_SAFE = ((V + 127) // 128) * 128 - CHUNK
    DUMP = B * D
    OUT_SZ = DUMP + NW * D * GROUP
    MCAP = B + _L

    tableT = table.T
    mesh = plsc.VectorSubcoreMesh(core_axis_name="c", subcore_axis_name="s")

    @functools.partial(
        pl.kernel,
        mesh=mesh,
        out_type=jax.ShapeDtypeStruct((OUT_SZ,), jnp.float32),
        scratch_types=[
            pltpu.VMEM((B,), jnp.int32),
            pltpu.VMEM((MCAP,), jnp.int32),
            pltpu.VMEM((MCAP,), jnp.int32),
            pltpu.VMEM((NCHUNK,), jnp.int32),
            pltpu.VMEM((NCHUNK,), jnp.int32),
            pltpu.VMEM((NCHUNK,), jnp.int32),
            pltpu.VMEM((2, D, CHUNK), jnp.float32),
            pltpu.VMEM((GROUP * D,), jnp.float32),
            pltpu.VMEM((GROUP * D,), jnp.int32),
            pltpu.VMEM((GROUP * D,), jnp.float32),
            pltpu.VMEM((GROUP * D,), jnp.int32),
            pltpu.SemaphoreType.DMA((2,)),
            pltpu.SemaphoreType.DMA,
            pltpu.SemaphoreType.DMA,
        ],
        compiler_params=pltpu.CompilerParams(needs_layout_passes=False),
    )
    def emb(idx_hbm, tab_hbm, out_hbm, xbuf, mx, mi, hist_v, base_v, cur_v,
            cbuf, vbuf_a, obuf_a, vbuf_b, obuf_b, fsem, ssem_a, ssem_b):
        wid = lax.axis_index("s") * NC + lax.axis_index("c")
        zeros = jnp.zeros((_L,), jnp.int32)
        iota = lax.iota(jnp.int32, _L)

        pltpu.sync_copy(idx_hbm, xbuf)
        hist_v[pl.ds(0, _L)] = zeros
        hist_v[pl.ds(_L, _L)] = zeros

        def scan_a(g, _):
            v = xbuf[pl.ds(pl.multiple_of(g * _L, _L), _L)]
            m = lax.shift_right_logical(v, RSHIFT) == wid
            t = lax.bitwise_and(lax.shift_right_logical(v, CSHIFT),
                                NCHUNK - 1)
            rank, last = plsc.scan_count(t, mask=m)
            plsc.addupdate_scatter(hist_v, [t], rank + 1, mask=last)
            return 0

        lax.fori_loop(0, B // _L, scan_a, 0)

        h0 = hist_v[pl.ds(0, _L)]
        c0 = plsc.cumsum(h0)
        h1 = hist_v[pl.ds(_L, _L)]
        c1 = plsc.cumsum(h1) + jnp.full((_L,), c0[_L - 1], jnp.int32)
        base_v[pl.ds(0, _L)] = c0 - h0
        base_v[pl.ds(_L, _L)] = c1 - h1
        cur_v[pl.ds(0, _L)] = c0 - h0
        cur_v[pl.ds(_L, _L)] = c1 - h1

        def scan_b(g, _):
            v = xbuf[pl.ds(pl.multiple_of(g * _L, _L), _L)]
            m = lax.shift_right_logical(v, RSHIFT) == wid
            t = lax.bitwise_and(lax.shift_right_logical(v, CSHIFT),
                                NCHUNK - 1)
            rank, last = plsc.scan_count(t, mask=m)
            slot = plsc.load_gather(cur_v, [t]) + rank
            plsc.store_scatter(mx, [slot], v, mask=m)
            plsc.store_scatter(mi, [slot], iota + g * _L, mask=m)
            plsc.addupdate_scatter(cur_v, [t], rank + 1, mask=last)
            return 0

        lax.fori_loop(0, B // _L, scan_b, 0)

        def chunk_base(c):
            return jnp.minimum(wid * RANGE + c * CHUNK, LAST_SAFE)

        def fire(c):
            p = pl.multiple_of(chunk_base(c), 128)
            pltpu.async_copy(tab_hbm.at[:, pl.ds(p, CHUNK)],
                             cbuf.at[lax.rem(c, 2)],
                             fsem.at[lax.rem(c, 2)])

        def drain(c):
            pltpu.make_async_copy(tab_hbm.at[:, pl.ds(0, CHUNK)],
                                  cbuf.at[lax.rem(c, 2)],
                                  fsem.at[lax.rem(c, 2)]).wait()

        def drain_scatter(vb, sm):
            pltpu.make_async_copy(out_hbm.at[pl.ds(0, GROUP * D)],
                                  vb, sm).wait()

        fire(0)

        dump_base = DUMP + wid * D * GROUP

        def step(c, gcount):
            @pl.when(c + 1 < NCHUNK)
            def _():
                fire(c + 1)

            drain(c)
            cpar = lax.rem(c, 2)
            cnt = plsc.load_gather(hist_v, [jnp.full((_L,), c, jnp.int32)])[0]
            bas = plsc.load_gather(base_v, [jnp.full((_L,), c, jnp.int32)])[0]
            cbase = chunk_base(c)
            ngroups = lax.shift_right_logical(cnt + (_L - 1), 4)

            def group(j, _):
                gj = gcount + j
                par = lax.rem(gj, 2)

                sel = (j * _L + iota) < cnt
                mm = bas + j * _L + iota
                mvx = plsc.load_gather(mx, [mm])
                mvi = plsc.load_gather(mi, [mm])
                l = jnp.clip(mvx - cbase, 0, CHUNK - 1)
                pv = jnp.full((_L,), cpar, jnp.int32)

                def body(vb, ob, sm):
                    del sm
                    for c0 in range(D):
                        vals = plsc.load_gather(
                            cbuf, [pv, jnp.full((_L,), c0, jnp.int32), l])
                        vb[pl.ds(c0 * _L, _L)] = vals
                        oidx = jnp.where(sel, mvi * D + c0,
                                         dump_base + c0 * _L + iota)
                        ob[pl.ds(c0 * _L, _L)] = oidx
                    pltpu.sync_copy(vb, out_hbm.at[ob])

                @pl.when(par == 0)
                def _():
                    body(vbuf_a, obuf_a, ssem_a)

                @pl.when(par == 1)
                def _():
                    body(vbuf_b, obuf_b, ssem_b)

                return 0

            lax.fori_loop(0, ngroups, group, 0)
            return gcount + ngroups

        lax.fori_loop(0, NCHUNK, step, jnp.int32(0))

    out_flat = emb(x.astype(jnp.int32), tableT)
    return out_flat[:B * D].reshape(B, D)


# speedup vs baseline: 52.0681x; 52.0681x over previous
import functools

import jax
import jax.numpy as jnp
from jax import lax
from jax.experimental import pallas as pl
from jax.experimental.pallas import tpu as pltpu
from jax.experimental.pallas import tpu_sc as plsc

_LANES = 16


def kernel(x, table):
    (B,) = x.shape
    V, D = table.shape
    info = plsc.get_sparse_core_info()
    NC, NS = info.num_cores, info.num_subcores
    NW = NC * NS
    b_per_w = B // NW
    WAVE = 8
    n_waves = b_per_w // WAVE

    tableT = table.T
    mesh = plsc.VectorSubcoreMesh(core_axis_name="c", subcore_axis_name="s")

    @functools.partial(
        pl.kernel,
        mesh=mesh,
        out_type=jax.ShapeDtypeStruct((D, B), jnp.float32),
        scratch_types=[
            pltpu.VMEM((b_per_w,), jnp.int32),
            pltpu.VMEM((2, WAVE, D, 128), jnp.float32),
            pltpu.VMEM((D, b_per_w), jnp.float32),
            pltpu.SemaphoreType.DMA((2,)),
        ],
        compiler_params=pltpu.CompilerParams(needs_layout_passes=False),
    )
    def emb(idx_hbm, tab_hbm, out_hbm, idx_v, dbuf, panel_v, sem):
        wid = lax.axis_index("s") * NC + lax.axis_index("c")
        base = wid * b_per_w
        pltpu.sync_copy(idx_hbm.at[pl.ds(base, b_per_w)], idx_v)

        def wave_idx(w, k):
            sel = plsc.load_gather(
                idx_v, [jnp.full((_LANES,), w * WAVE + k, jnp.int32)])
            return sel, sel[0]

        def fire_wave(w):
            def fire(k, _):
                _, r = wave_idx(w, k)
                p = pl.multiple_of(
                    lax.shift_right_logical(r, 7) * 128, 128)
                pltpu.async_copy(tab_hbm.at[:, pl.ds(p, 128)],
                                 dbuf.at[lax.rem(w, 2), k],
                                 sem.at[lax.rem(w, 2)])
                return 0
            lax.fori_loop(0, WAVE, fire, 0)

        def drain_wave(w):
            pltpu.make_async_copy(tab_hbm.at[:, pl.ds(0, WAVE * 128)],
                                  dbuf.at[lax.rem(w, 2)],
                                  sem.at[lax.rem(w, 2)]).wait()

        fire_wave(0)

        def step(w, _):
            @pl.when(w + 1 < n_waves)
            def _():
                fire_wave(w + 1)

            drain_wave(w)

            def extract(k, _):
                rvec, r = wave_idx(w, k)
                lane = lax.bitwise_and(rvec, 127)
                kk = jnp.full((_LANES,), k, jnp.int32)
                ww = jnp.full((_LANES,), lax.rem(w, 2), jnp.int32)
                ii = jnp.full((_LANES,), w * WAVE + k, jnp.int32)
                for c0 in range(0, D, _LANES):
                    cs = lax.iota(jnp.int32, _LANES) + c0
                    vals = plsc.load_gather(dbuf, [ww, kk, cs, lane])
                    plsc.store_scatter(panel_v, [cs, ii], vals)
                return 0

            lax.fori_loop(0, WAVE, extract, 0)
            return 0

        lax.fori_loop(0, n_waves, step, 0)
        pltpu.sync_copy(panel_v, out_hbm.at[:, pl.ds(base, b_per_w)])

    outT = emb(x.astype(jnp.int32), tableT)
    return outT.T
